# baseline (speedup 1.0000x reference)
import functools

import jax
import jax.numpy as jnp
from jax import lax
from jax.experimental import pallas as pl
from jax.experimental.pallas import tpu as pltpu
from jax.experimental.pallas import tpu_sc as plsc

N = 10000
E = 320000
F = 128
NC = 2
NS = 16
NW = NC * NS
EPW = E // NW
DCH = 128
DNCH = 80
CCH = 128
CNCH = 80
DST0 = 128
NP = 10112
RPT = NP // NS
DW = 8
NBUF = 4

_MESH = plsc.VectorSubcoreMesh(core_axis_name="c", subcore_axis_name="s")


def _sc_deg_body(edges_hbm, out_hbm, idx_v, ones_v, zbuf, acc, ssem):
    c = lax.axis_index("c")
    s = lax.axis_index("s")
    wid = s * NC + c

    @pl.loop(0, DCH)
    def _fill(i):
        for k in range(F // 16):
            ones_v[i, pl.ds(k * 16, 16)] = jnp.full((16,), 1.0, jnp.float32)
            zbuf[i, pl.ds(k * 16, 16)] = jnp.zeros((16,), jnp.float32)

    for a in range(4):
        for k in range(RPT // DCH):
            pltpu.sync_copy(zbuf, acc.at[pl.ds(s * RPT + k * DCH, DCH)])
        pltpu.sync_copy(zbuf.at[pl.ds(0, RPT % DCH)],
                        acc.at[pl.ds(s * RPT + (RPT // DCH) * DCH,
                                     RPT % DCH)])
        pltpu.sync_copy(edges_hbm.at[a, wid], idx_v)
        plsc.subcore_barrier()

        @pl.loop(0, DNCH)
        def _chunk(j):
            pltpu.sync_copy(ones_v, acc.at[idx_v.at[j]], add=True)

        plsc.subcore_barrier()
        pltpu.sync_copy(acc.at[pl.ds(s * RPT, RPT)],
                        out_hbm.at[a, c, pl.ds(s * RPT, RPT)])
        plsc.subcore_barrier()


_sc_degrees = pl.kernel(
    _sc_deg_body,
    out_type=jax.ShapeDtypeStruct((4, NC, NP, F), jnp.float32),
    mesh=_MESH,
    scratch_types=[
        pltpu.VMEM((DNCH, DCH), jnp.int32),
        pltpu.VMEM((DCH, F), jnp.float32),
        pltpu.VMEM((DCH, F), jnp.float32),
        pltpu.VMEM_SHARED((NP, F), jnp.float32),
        pltpu.SemaphoreType.DMA,
    ],
)


def _sc_conv2_body(ha_hbm, hb_hbm, srca_hbm, dsta_hbm, srcb_hbm, dstb_hbm,
                   outa_hbm, outb_hbm,
                   idx_v, rows_v, gsem, acc):
    c = lax.axis_index("c")
    s = lax.axis_index("s")
    wid = s * NC + c

    for h_hbm, src_hbm, dst_hbm, out_hbm in (
            (ha_hbm, srca_hbm, dsta_hbm, outa_hbm),
            (hb_hbm, srcb_hbm, dstb_hbm, outb_hbm)):
        @pl.loop(0, CCH)
        def _zrow(i):
            for k in range(F // 16):
                rows_v[i, pl.ds(k * 16, 16)] = jnp.zeros((16,), jnp.float32)

        pltpu.sync_copy(src_hbm.at[wid], idx_v.at[pl.ds(0, CNCH)])
        pltpu.sync_copy(dst_hbm.at[wid], idx_v.at[pl.ds(DST0, CNCH)])
        for k in range(RPT // CCH):
            pltpu.sync_copy(rows_v, acc.at[pl.ds(s * RPT + k * CCH, CCH)])
        pltpu.sync_copy(rows_v.at[pl.ds(0, RPT % CCH)],
                        acc.at[pl.ds(s * RPT + (RPT // CCH) * CCH,
                                     RPT % CCH)])
        plsc.subcore_barrier()

        @pl.loop(0, CNCH)
        def _blk(j):
            pltpu.async_copy(h_hbm.at[idx_v.at[j]], rows_v, gsem).wait()
            pltpu.sync_copy(rows_v, acc.at[idx_v.at[DST0 + j]], add=True)

        plsc.subcore_barrier()
        pltpu.sync_copy(acc.at[pl.ds(s * RPT, RPT)],
                        out_hbm.at[c, pl.ds(s * RPT, RPT)])
        plsc.subcore_barrier()


_sc_conv2 = pl.kernel(
    _sc_conv2_body,
    out_type=[jax.ShapeDtypeStruct((NC, NP, F), jnp.float32),
              jax.ShapeDtypeStruct((NC, NP, F), jnp.float32)],
    mesh=_MESH,
    scratch_types=[
        pltpu.VMEM((DST0 + CNCH, CCH), jnp.int32),
        pltpu.VMEM((CCH, F), jnp.float32),
        pltpu.SemaphoreType.DMA,
        pltpu.VMEM_SHARED((NP, F), jnp.float32),
    ],
)


_BR = 2000


def _mm_body(x_ref, degp_ref, w_ref, o_ref):
    deg = degp_ref[0, :, 0:1] + degp_ref[1, :, 0:1]
    dinv = lax.rsqrt(jnp.maximum(deg, 1.0))
    o_ref[...] = jnp.dot(x_ref[...] * dinv, w_ref[...],
                         preferred_element_type=jnp.float32)


@functools.cache
def _make_mm(d):
    return pl.pallas_call(
        _mm_body,
        grid=(N // _BR,),
        in_specs=[
            pl.BlockSpec((_BR, d), lambda i: (i, 0)),
            pl.BlockSpec((NC, _BR, DW), lambda i: (0, i, 0)),
            pl.BlockSpec((d, F), lambda i: (0, 0)),
        ],
        out_specs=pl.BlockSpec((_BR, F), lambda i: (i, 0)),
        out_shape=jax.ShapeDtypeStruct((N, F), jnp.float32),
    )


def _post_body(p_ref, degp_ref, b_ref, g_ref, bb_ref, o_ref):
    x = p_ref[0] + p_ref[1]
    deg = degp_ref[0, :, 0:1] + degp_ref[1, :, 0:1]
    x = x * lax.rsqrt(jnp.maximum(deg, 1.0)) + b_ref[...]
    mu = jnp.mean(x, axis=-1, keepdims=True)
    var = jnp.mean((x - mu) ** 2, axis=-1, keepdims=True)
    y = (x - mu) * lax.rsqrt(var + 1e-5) * g_ref[...] + bb_ref[...]
    o_ref[...] = jnp.where(y > 0.0, y, jnp.exp(jnp.minimum(y, 0.0)) - 1.0)


_post = pl.pallas_call(
    _post_body,
    grid=(N // _BR,),
    in_specs=[
        pl.BlockSpec((NC, _BR, F), lambda i: (0, i, 0)),
        pl.BlockSpec((NC, _BR, DW), lambda i: (0, i, 0)),
        pl.BlockSpec((1, F), lambda i: (0, 0)),
        pl.BlockSpec((1, F), lambda i: (0, 0)),
        pl.BlockSpec((1, F), lambda i: (0, 0)),
    ],
    out_specs=pl.BlockSpec((_BR, F), lambda i: (i, 0)),
    out_shape=jax.ShapeDtypeStruct((N, F), jnp.float32),
)


def _readout_body(hn, hl, wm1, bm1, wm2, bm2, wd1, bd1, wd2, bd2, wd3, bd3,
                  wp1, bp1, wp2, bp2, wp3, bp3, rec_ref, prop_ref, z_ref):
    hg = jnp.mean(hn[...], axis=0, keepdims=True) \
        + jnp.mean(hl[...], axis=0, keepdims=True)
    hg8 = jnp.broadcast_to(hg, (8, F))

    def dot(a, b):
        return jnp.dot(a, b, preferred_element_type=jnp.float32)

    t = jnp.maximum(dot(hg8, wm1[...]) + bm1[...], 0.0)
    z = dot(t, wm2[...]) + bm2[...]
    d = jnp.maximum(dot(z, wd1[...]) + bd1[...], 0.0)
    d = jnp.maximum(dot(d, wd2[...]) + bd2[...], 0.0)
    rec = dot(d, wd3[...]) + bd3[...]
    p = jnp.maximum(dot(z, wp1[...]) + bp1[...], 0.0)
    p = jnp.maximum(dot(p, wp2[...]) + bp2[...], 0.0)
    prop = dot(p, wp3[...]) + bp3[...]
    rec_ref[...] = rec
    prop_ref[...] = prop
    z_ref[...] = z


_readout = pl.pallas_call(
    _readout_body,
    out_shape=(
        jax.ShapeDtypeStruct((8, 64), jnp.float32),
        jax.ShapeDtypeStruct((8, F), jnp.float32),
        jax.ShapeDtypeStruct((8, 64), jnp.float32),
    ),
)


def kernel(x_l, x_n, edge_l2n, edge_n2l, params):
    f32 = jnp.float32
    i32 = jnp.int32

    def pad_edges(arr, padvals, nch, ch):
        a = arr.astype(i32).reshape(NW, EPW)
        p = jnp.broadcast_to(padvals[:, None], (NW, nch * ch - EPW))
        return jnp.concatenate([a, p], axis=1).reshape(NW, nch, ch)

    widv = jnp.arange(NW, dtype=i32)
    pad_lo = (widv * 313) % N
    pad_hi = N + widv

    sl, dl = edge_l2n[0], edge_l2n[1]
    sn, dn = edge_n2l[0], edge_n2l[1]
    src_l2n = pad_edges(sl, pad_lo, CNCH, CCH)
    dst_l2n = pad_edges(dl, pad_hi, CNCH, CCH)
    src_n2l = pad_edges(sn, pad_lo, CNCH, CCH)
    dst_n2l = pad_edges(dn, pad_hi, CNCH, CCH)
    deg_edges = jnp.stack(
        [pad_edges(x, pad_hi, DNCH, DCH) for x in (sl, dl, sn, dn)])

    degs = _sc_degrees(deg_edges)[..., 0:DW]
    dp_sl, dp_dl, dp_sn, dp_dn = degs[0], degs[1], degs[2], degs[3]

    def r1(v):
        return v.reshape(1, -1)

    h_l, h_n = x_l, x_n
    for i in range(3):
        mm_l = _make_mm(h_l.shape[1])
        mm_n = _make_mm(h_n.shape[1])
        hs_l2n = mm_l(h_l, dp_sl, params['W_l2n'][i])
        hs_n2l = mm_n(h_n, dp_sn, params['W_n2l'][i])
        agg_n, agg_l = _sc_conv2(hs_l2n, hs_n2l, src_l2n, dst_l2n,
                                 src_n2l, dst_n2l)
        h_n = _post(agg_n, dp_dl, r1(params['b_l2n'][i]),
                    r1(params['ln_g_n'][i]), r1(params['ln_b_n'][i]))
        h_l = _post(agg_l, dp_dn, r1(params['b_n2l'][i]),
                    r1(params['ln_g_l'][i]), r1(params['ln_b_l'][i]))

    wp3 = jnp.pad(params['Wp3'], ((0, 0), (0, F - 1)))
    bp3 = jnp.pad(r1(params['bp3']), ((0, 0), (0, F - 1)))
    rec8, prop8, z8 = _readout(
        h_n, h_l,
        params['Wm1'], r1(params['bm1']), params['Wm2'], r1(params['bm2']),
        params['Wd1'], r1(params['bd1']), params['Wd2'], r1(params['bd2']),
        params['Wd3'], r1(params['bd3']),
        params['Wp1'], r1(params['bp1']), params['Wp2'], r1(params['bp2']),
        wp3, bp3)
    return rec8[0:1, :], prop8[0:1, 0:1], z8[0:1, :]


# speedup vs baseline: 1.0007x; 1.0007x over previous
import functools

import jax
import jax.numpy as jnp
from jax import lax
from jax.experimental import pallas as pl
from jax.experimental.pallas import tpu as pltpu
from jax.experimental.pallas import tpu_sc as plsc

N = 10000
E = 320000
F = 128
NC = 2
NS = 16
NW = NC * NS
EPW = E // NW
DCH = 128
DNCH = 80
CCH = 128
CNCH = 80
DST0 = 128
NP = 10112
RPT = NP // NS
DW = 8
NBUF = 4

_MESH = plsc.VectorSubcoreMesh(core_axis_name="c", subcore_axis_name="s")


def _sc_deg_body(edges_hbm, out_hbm, idx_v, ones_v, zbuf, acc,
                 d0, d1, d2, d3, d4, d5, d6, d7):
    dsems = (d0, d1, d2, d3, d4, d5, d6, d7)
    c = lax.axis_index("c")
    s = lax.axis_index("s")
    wid = s * NC + c

    @pl.loop(0, DCH)
    def _fill(i):
        for k in range(F // 16):
            ones_v[i, pl.ds(k * 16, 16)] = jnp.full((16,), 1.0, jnp.float32)
            zbuf[i, pl.ds(k * 16, 16)] = jnp.zeros((16,), jnp.float32)

    for a in range(4):
        for k in range(RPT // DCH):
            pltpu.sync_copy(zbuf, acc.at[pl.ds(s * RPT + k * DCH, DCH)])
        pltpu.sync_copy(zbuf.at[pl.ds(0, RPT % DCH)],
                        acc.at[pl.ds(s * RPT + (RPT // DCH) * DCH,
                                     RPT % DCH)])
        pltpu.sync_copy(edges_hbm.at[a, wid], idx_v)
        plsc.subcore_barrier()

        @pl.loop(0, DNCH // 8)
        def _chunk(m):
            j0 = m * 8
            ds = [pltpu.async_copy(ones_v, acc.at[idx_v.at[j0 + k]],
                                   dsems[k], add=True) for k in range(8)]
            for d in ds:
                d.wait()

        plsc.subcore_barrier()
        pltpu.sync_copy(acc.at[pl.ds(s * RPT, RPT)],
                        out_hbm.at[a, c, pl.ds(s * RPT, RPT)])
        plsc.subcore_barrier()


_sc_degrees = pl.kernel(
    _sc_deg_body,
    out_type=jax.ShapeDtypeStruct((4, NC, NP, F), jnp.float32),
    mesh=_MESH,
    scratch_types=[
        pltpu.VMEM((DNCH, DCH), jnp.int32),
        pltpu.VMEM((DCH, F), jnp.float32),
        pltpu.VMEM((DCH, F), jnp.float32),
        pltpu.VMEM_SHARED((NP, F), jnp.float32),
        pltpu.SemaphoreType.DMA,
        pltpu.SemaphoreType.DMA,
        pltpu.SemaphoreType.DMA,
        pltpu.SemaphoreType.DMA,
        pltpu.SemaphoreType.DMA,
        pltpu.SemaphoreType.DMA,
        pltpu.SemaphoreType.DMA,
        pltpu.SemaphoreType.DMA,
    ],
)


def _sc_conv2_body(ha_hbm, hb_hbm, srca_hbm, dsta_hbm, srcb_hbm, dstb_hbm,
                   outa_hbm, outb_hbm,
                   idx_v, rows_v, gsem, acc):
    c = lax.axis_index("c")
    s = lax.axis_index("s")
    wid = s * NC + c

    for h_hbm, src_hbm, dst_hbm, out_hbm in (
            (ha_hbm, srca_hbm, dsta_hbm, outa_hbm),
            (hb_hbm, srcb_hbm, dstb_hbm, outb_hbm)):
        @pl.loop(0, CCH)
        def _zrow(i):
            for k in range(F // 16):
                rows_v[i, pl.ds(k * 16, 16)] = jnp.zeros((16,), jnp.float32)

        pltpu.sync_copy(src_hbm.at[wid], idx_v.at[pl.ds(0, CNCH)])
        pltpu.sync_copy(dst_hbm.at[wid], idx_v.at[pl.ds(DST0, CNCH)])
        for k in range(RPT // CCH):
            pltpu.sync_copy(rows_v, acc.at[pl.ds(s * RPT + k * CCH, CCH)])
        pltpu.sync_copy(rows_v.at[pl.ds(0, RPT % CCH)],
                        acc.at[pl.ds(s * RPT + (RPT // CCH) * CCH,
                                     RPT % CCH)])
        plsc.subcore_barrier()

        @pl.loop(0, CNCH)
        def _blk(j):
            pltpu.async_copy(h_hbm.at[idx_v.at[j]], rows_v, gsem).wait()
            pltpu.sync_copy(rows_v, acc.at[idx_v.at[DST0 + j]], add=True)

        plsc.subcore_barrier()
        pltpu.sync_copy(acc.at[pl.ds(s * RPT, RPT)],
                        out_hbm.at[c, pl.ds(s * RPT, RPT)])
        plsc.subcore_barrier()


_sc_conv2 = pl.kernel(
    _sc_conv2_body,
    out_type=[jax.ShapeDtypeStruct((NC, NP, F), jnp.float32),
              jax.ShapeDtypeStruct((NC, NP, F), jnp.float32)],
    mesh=_MESH,
    scratch_types=[
        pltpu.VMEM((DST0 + CNCH, CCH), jnp.int32),
        pltpu.VMEM((CCH, F), jnp.float32),
        pltpu.SemaphoreType.DMA,
        pltpu.VMEM_SHARED((NP, F), jnp.float32),
    ],
)


_BR = 2000


def _mm_body(x_ref, degp_ref, w_ref, o_ref):
    deg = degp_ref[0, :, 0:1] + degp_ref[1, :, 0:1]
    dinv = lax.rsqrt(jnp.maximum(deg, 1.0))
    o_ref[...] = jnp.dot(x_ref[...] * dinv, w_ref[...],
                         preferred_element_type=jnp.float32)


@functools.cache
def _make_mm(d):
    return pl.pallas_call(
        _mm_body,
        grid=(N // _BR,),
        in_specs=[
            pl.BlockSpec((_BR, d), lambda i: (i, 0)),
            pl.BlockSpec((NC, _BR, DW), lambda i: (0, i, 0)),
            pl.BlockSpec((d, F), lambda i: (0, 0)),
        ],
        out_specs=pl.BlockSpec((_BR, F), lambda i: (i, 0)),
        out_shape=jax.ShapeDtypeStruct((N, F), jnp.float32),
    )


def _post_body(p_ref, degp_ref, b_ref, g_ref, bb_ref, o_ref):
    x = p_ref[0] + p_ref[1]
    deg = degp_ref[0, :, 0:1] + degp_ref[1, :, 0:1]
    x = x * lax.rsqrt(jnp.maximum(deg, 1.0)) + b_ref[...]
    mu = jnp.mean(x, axis=-1, keepdims=True)
    var = jnp.mean((x - mu) ** 2, axis=-1, keepdims=True)
    y = (x - mu) * lax.rsqrt(var + 1e-5) * g_ref[...] + bb_ref[...]
    o_ref[...] = jnp.where(y > 0.0, y, jnp.exp(jnp.minimum(y, 0.0)) - 1.0)


_post = pl.pallas_call(
    _post_body,
    grid=(N // _BR,),
    in_specs=[
        pl.BlockSpec((NC, _BR, F), lambda i: (0, i, 0)),
        pl.BlockSpec((NC, _BR, DW), lambda i: (0, i, 0)),
        pl.BlockSpec((1, F), lambda i: (0, 0)),
        pl.BlockSpec((1, F), lambda i: (0, 0)),
        pl.BlockSpec((1, F), lambda i: (0, 0)),
    ],
    out_specs=pl.BlockSpec((_BR, F), lambda i: (i, 0)),
    out_shape=jax.ShapeDtypeStruct((N, F), jnp.float32),
)


def _readout_body(hn, hl, wm1, bm1, wm2, bm2, wd1, bd1, wd2, bd2, wd3, bd3,
                  wp1, bp1, wp2, bp2, wp3, bp3, rec_ref, prop_ref, z_ref):
    hg = jnp.mean(hn[...], axis=0, keepdims=True) \
        + jnp.mean(hl[...], axis=0, keepdims=True)
    hg8 = jnp.broadcast_to(hg, (8, F))

    def dot(a, b):
        return jnp.dot(a, b, preferred_element_type=jnp.float32)

    t = jnp.maximum(dot(hg8, wm1[...]) + bm1[...], 0.0)
    z = dot(t, wm2[...]) + bm2[...]
    d = jnp.maximum(dot(z, wd1[...]) + bd1[...], 0.0)
    d = jnp.maximum(dot(d, wd2[...]) + bd2[...], 0.0)
    rec = dot(d, wd3[...]) + bd3[...]
    p = jnp.maximum(dot(z, wp1[...]) + bp1[...], 0.0)
    p = jnp.maximum(dot(p, wp2[...]) + bp2[...], 0.0)
    prop = dot(p, wp3[...]) + bp3[...]
    rec_ref[...] = rec
    prop_ref[...] = prop
    z_ref[...] = z


_readout = pl.pallas_call(
    _readout_body,
    out_shape=(
        jax.ShapeDtypeStruct((8, 64), jnp.float32),
        jax.ShapeDtypeStruct((8, F), jnp.float32),
        jax.ShapeDtypeStruct((8, 64), jnp.float32),
    ),
)


def kernel(x_l, x_n, edge_l2n, edge_n2l, params):
    f32 = jnp.float32
    i32 = jnp.int32

    def pad_edges(arr, padvals, nch, ch):
        a = arr.astype(i32).reshape(NW, EPW)
        p = jnp.broadcast_to(padvals[:, None], (NW, nch * ch - EPW))
        return jnp.concatenate([a, p], axis=1).reshape(NW, nch, ch)

    widv = jnp.arange(NW, dtype=i32)
    pad_lo = (widv * 313) % N
    pad_hi = N + widv

    sl, dl = edge_l2n[0], edge_l2n[1]
    sn, dn = edge_n2l[0], edge_n2l[1]
    src_l2n = pad_edges(sl, pad_lo, CNCH, CCH)
    dst_l2n = pad_edges(dl, pad_hi, CNCH, CCH)
    src_n2l = pad_edges(sn, pad_lo, CNCH, CCH)
    dst_n2l = pad_edges(dn, pad_hi, CNCH, CCH)
    deg_edges = jnp.stack(
        [pad_edges(x, pad_hi, DNCH, DCH) for x in (sl, dl, sn, dn)])

    degs = _sc_degrees(deg_edges)[..., 0:DW]
    dp_sl, dp_dl, dp_sn, dp_dn = degs[0], degs[1], degs[2], degs[3]

    def r1(v):
        return v.reshape(1, -1)

    h_l, h_n = x_l, x_n
    for i in range(3):
        mm_l = _make_mm(h_l.shape[1])
        mm_n = _make_mm(h_n.shape[1])
        hs_l2n = mm_l(h_l, dp_sl, params['W_l2n'][i])
        hs_n2l = mm_n(h_n, dp_sn, params['W_n2l'][i])
        agg_n, agg_l = _sc_conv2(hs_l2n, hs_n2l, src_l2n, dst_l2n,
                                 src_n2l, dst_n2l)
        h_n = _post(agg_n, dp_dl, r1(params['b_l2n'][i]),
                    r1(params['ln_g_n'][i]), r1(params['ln_b_n'][i]))
        h_l = _post(agg_l, dp_dn, r1(params['b_n2l'][i]),
                    r1(params['ln_g_l'][i]), r1(params['ln_b_l'][i]))

    wp3 = jnp.pad(params['Wp3'], ((0, 0), (0, F - 1)))
    bp3 = jnp.pad(r1(params['bp3']), ((0, 0), (0, F - 1)))
    rec8, prop8, z8 = _readout(
        h_n, h_l,
        params['Wm1'], r1(params['bm1']), params['Wm2'], r1(params['bm2']),
        params['Wd1'], r1(params['bd1']), params['Wd2'], r1(params['bd2']),
        params['Wd3'], r1(params['bd3']),
        params['Wp1'], r1(params['bp1']), params['Wp2'], r1(params['bp2']),
        wp3, bp3)
    return rec8[0:1, :], prop8[0:1, 0:1], z8[0:1, :]
